# Initial kernel scaffold; baseline (speedup 1.0000x reference)
import jax
import jax.numpy as jnp
from jax.experimental import pallas as pl


def kernel(features_1, edge_index_1, features_2, edge_index_2, W1, b1, W2, b2):
    raise NotImplementedError("write your pallas kernel here")


# speedup vs baseline: 26.6015x; 26.6015x over previous
import functools

import jax
import jax.numpy as jnp
from jax import lax
from jax.experimental import pallas as pl
from jax.experimental.pallas import tpu as pltpu
from jax.experimental.pallas import tpu_sc as plsc

N = 100000
E = 3200000
NP = 102400
SLICE = NP // 16
R0 = E // 128
RPW = 784
R3 = RPW * 32
K = 8
NB_ROWS = RPW // K
F32 = jnp.float32

_mesh = plsc.VectorSubcoreMesh(core_axis_name="c", subcore_axis_name="s")
_sc_params = pltpu.CompilerParams(use_tc_tiling_on_sc=False)


def _sc_deg_body(dst_hbm, zeros1_hbm, ones_hbm, deg_out, deg_sh, dbuf, ones_v):
    cid = lax.axis_index("c")
    sid = lax.axis_index("s")
    wid = sid * 2 + cid
    nsl = pl.ds(sid * SLICE, SLICE)
    pltpu.sync_copy(zeros1_hbm, deg_sh.at[nsl])
    pltpu.sync_copy(ones_hbm, ones_v)
    plsc.subcore_barrier()

    row0 = wid * RPW

    def step(r, _):
        base = row0 + r * K
        pltpu.sync_copy(dst_hbm.at[pl.ds(base, K)], dbuf)
        for j in range(K):
            pltpu.sync_copy(ones_v, deg_sh.at[dbuf.at[j]], add=True)
        return _

    lax.fori_loop(0, NB_ROWS, step, None)
    plsc.subcore_barrier()
    pltpu.sync_copy(deg_sh.at[nsl], deg_out.at[cid, nsl])


def _sc_deg(dst_rows, zeros1, ones128):
    return pl.kernel(
        _sc_deg_body,
        out_type=jax.ShapeDtypeStruct((2, NP), F32),
        mesh=_mesh,
        compiler_params=_sc_params,
        scratch_types=[
            pltpu.VMEM_SHARED((NP,), F32),
            pltpu.VMEM((K, 128), jnp.int32),
            pltpu.VMEM((128,), F32),
        ],
    )(dst_rows, zeros1, ones128)


def _sc_main_body(src_hbm, dst_hbm, ya_hbm, yb_hbm, dinv_hbm, zeros16_hbm,
                  zeros1_hbm, agg_a_out, agg_b_out, t_out,
                  acc_sh, t_sh, dinv_sh, sbuf, dbuf, rbuf, dvbuf, sem, sem2):
    cid = lax.axis_index("c")
    sid = lax.axis_index("s")
    wid = sid * 2 + cid
    nsl = pl.ds(sid * SLICE, SLICE)
    row0 = wid * RPW

    pltpu.sync_copy(dinv_hbm.at[nsl], dinv_sh.at[nsl])
    pltpu.sync_copy(zeros16_hbm, acc_sh.at[nsl])
    pltpu.sync_copy(zeros1_hbm, t_sh.at[nsl])
    plsc.subcore_barrier()

    def step_a(r, _):
        base = row0 + r * K
        pltpu.sync_copy(src_hbm.at[pl.ds(base, K)], sbuf)
        pltpu.sync_copy(dst_hbm.at[pl.ds(base, K)], dbuf)
        for j in range(K):
            pltpu.async_copy(ya_hbm.at[sbuf.at[j]], rbuf, sem).wait()
            pltpu.sync_copy(rbuf, acc_sh.at[dbuf.at[j]], add=True)
            pltpu.async_copy(dinv_sh.at[dbuf.at[j]], dvbuf, sem2).wait()
            pltpu.sync_copy(dvbuf, t_sh.at[sbuf.at[j]], add=True)
        return _

    lax.fori_loop(0, NB_ROWS, step_a, None)
    plsc.subcore_barrier()
    pltpu.sync_copy(acc_sh.at[nsl], agg_a_out.at[cid, nsl])
    pltpu.sync_copy(t_sh.at[nsl], t_out.at[cid, nsl])
    pltpu.sync_copy(zeros16_hbm, acc_sh.at[nsl])
    plsc.subcore_barrier()

    def step_b(r, _):
        base = row0 + r * K
        pltpu.sync_copy(src_hbm.at[pl.ds(base, K)], sbuf)
        pltpu.sync_copy(dst_hbm.at[pl.ds(base, K)], dbuf)
        for j in range(K):
            pltpu.async_copy(yb_hbm.at[sbuf.at[j]], rbuf, sem).wait()
            pltpu.sync_copy(rbuf, acc_sh.at[dbuf.at[j]], add=True)
        return _

    lax.fori_loop(0, NB_ROWS, step_b, None)
    plsc.subcore_barrier()
    pltpu.sync_copy(acc_sh.at[nsl], agg_b_out.at[cid, nsl])


def _sc_main(src_rows, dst_rows, ya, yb, dinv, zeros16, zeros1):
    return pl.kernel(
        _sc_main_body,
        out_type=[
            jax.ShapeDtypeStruct((2, NP, 16), F32),
            jax.ShapeDtypeStruct((2, NP, 16), F32),
            jax.ShapeDtypeStruct((2, NP), F32),
        ],
        mesh=_mesh,
        compiler_params=_sc_params,
        scratch_types=[
            pltpu.VMEM_SHARED((NP, 16), F32),
            pltpu.VMEM_SHARED((NP,), F32),
            pltpu.VMEM_SHARED((NP,), F32),
            pltpu.VMEM((K, 128), jnp.int32),
            pltpu.VMEM((K, 128), jnp.int32),
            pltpu.VMEM((128, 16), F32),
            pltpu.VMEM((128,), F32),
            pltpu.SemaphoreType.DMA,
            pltpu.SemaphoreType.DMA,
        ],
    )(src_rows, dst_rows, ya, yb, dinv, zeros16, zeros1)


def _tc_prep_body(d0_ref, d1_ref, xa_ref, xb_ref, dinv_ref, ya_ref, yb_ref):
    d = d0_ref[0] + d1_ref[0] + 1.0
    dv = lax.rsqrt(d)
    dinv_ref[0] = dv
    ya_ref[0] = dv * xa_ref[0]
    yb_ref[0] = dv * xb_ref[0]


def _tc_prep(d0, d1, xa, xb):
    R = 1024
    nb = NP // R
    grid = (2, nb)
    col = pl.BlockSpec((1, R, 1), lambda g, i: (g, i, 0))
    mat = pl.BlockSpec((1, R, 16), lambda g, i: (g, i, 0))
    return pl.pallas_call(
        _tc_prep_body,
        grid=grid,
        in_specs=[col, col, mat, mat],
        out_specs=[col, mat, mat],
        out_shape=[
            jax.ShapeDtypeStruct((2, NP, 1), F32),
            jax.ShapeDtypeStruct((2, NP, 16), F32),
            jax.ShapeDtypeStruct((2, NP, 16), F32),
        ],
    )(d0, d1, xa, xb)


def _tc_final_body(aggA0, aggA1, aggB0, aggB1, xa, xb, dinv, t0, t1,
                   W1a, W1b, b1, W2, b2, out_ref, acc, *, nb, rows):
    g = pl.program_id(0)
    i = pl.program_id(1)

    @pl.when(i == 0)
    def _():
        acc[...] = jnp.zeros_like(acc)

    dv = dinv[0]
    aa = dv * (aggA0[0] + aggA1[0]) + (dv * dv) * xa[0]
    ab = dv * (aggB0[0] + aggB1[0]) + (dv * dv) * xb[0]
    h1 = jnp.maximum(
        jnp.dot(aa, W1a[...], preferred_element_type=F32)
        + jnp.dot(ab, W1b[...], preferred_element_type=F32)
        + b1[...], 0.0)
    rowid = i * rows + lax.broadcasted_iota(jnp.int32, (rows, 1), 0)
    m = jnp.where(rowid < N, 1.0, 0.0)
    c = m * dv * (t0[0] + t1[0] + dv)
    acc[...] += jnp.sum(c * h1, axis=0, keepdims=True)

    @pl.when(i == nb - 1)
    def _():
        out_ref[0] = (jnp.dot(acc[...], W2[...], preferred_element_type=F32)
                      * (1.0 / N) + b2[...])


def _tc_final(aggA0, aggA1, aggB0, aggB1, xa, xb, dinv, t0, t1,
              W1a, W1b, b1, W2, b2):
    R = 1024
    nb = NP // R
    grid = (2, nb)
    col = pl.BlockSpec((1, R, 1), lambda g, i: (g, i, 0))
    mat = pl.BlockSpec((1, R, 16), lambda g, i: (g, i, 0))
    full = lambda s: pl.BlockSpec(s, lambda g, i: tuple(0 for _ in s))
    body = functools.partial(_tc_final_body, nb=nb, rows=R)
    return pl.pallas_call(
        body,
        grid=grid,
        in_specs=[mat, mat, mat, mat, mat, mat, col, col, col,
                  full((16, 64)), full((16, 64)), full((1, 64)),
                  full((64, 32)), full((1, 32))],
        out_specs=pl.BlockSpec((1, 1, 32), lambda g, i: (g, 0, 0)),
        out_shape=jax.ShapeDtypeStruct((2, 1, 32), F32),
        scratch_shapes=[pltpu.VMEM((1, 64), F32)],
    )(aggA0, aggA1, aggB0, aggB1, xa, xb, dinv, t0, t1,
      W1a, W1b, b1, W2, b2)


def _pad_edges(v):
    pad = jnp.full((R3 * 128 - E,), N, dtype=jnp.int32)
    return jnp.concatenate([v, pad]).reshape(R3, 128)


def kernel(features_1, edge_index_1, features_2, edge_index_2, W1, b1, W2, b2):
    f32 = jnp.float32
    xs = []
    for f in (features_1, features_2):
        xp = jnp.pad(f, ((0, NP - N), (0, 32 - f.shape[1])))
        xs.append(xp)
    xa = jnp.stack([x[:, :16] for x in xs])
    xb = jnp.stack([x[:, 16:] for x in xs])
    src1, dst1 = _pad_edges(edge_index_1[0]), _pad_edges(edge_index_1[1])
    src2, dst2 = _pad_edges(edge_index_2[0]), _pad_edges(edge_index_2[1])
    zeros16 = jnp.zeros((SLICE, 16), f32)
    zeros1 = jnp.zeros((SLICE,), f32)
    ones128 = jnp.ones((128,), f32)
    W1p = jnp.pad(W1, ((0, 32 - W1.shape[0]), (0, 0)))
    W1a_, W1b_ = W1p[:16], W1p[16:]
    b1r, b2r = b1.reshape(1, 64), b2.reshape(1, 32)

    degp1 = _sc_deg(dst1, zeros1, ones128)
    degp2 = _sc_deg(dst2, zeros1, ones128)
    d0 = jnp.stack([degp1[0], degp2[0]])[..., None]
    d1 = jnp.stack([degp1[1], degp2[1]])[..., None]

    dinv_c, ya, yb = _tc_prep(d0, d1, xa, xb)
    dinv_flat = dinv_c[..., 0]

    aggA1_, aggB1_, t1_ = _sc_main(src1, dst1, ya[0], yb[0], dinv_flat[0],
                                   zeros16, zeros1)
    aggA2_, aggB2_, t2_ = _sc_main(src2, dst2, ya[1], yb[1], dinv_flat[1],
                                   zeros16, zeros1)

    aggA0 = jnp.stack([aggA1_[0], aggA2_[0]])
    aggA1 = jnp.stack([aggA1_[1], aggA2_[1]])
    aggB0 = jnp.stack([aggB1_[0], aggB2_[0]])
    aggB1 = jnp.stack([aggB1_[1], aggB2_[1]])
    t0 = jnp.stack([t1_[0], t2_[0]])[..., None]
    t1c = jnp.stack([t1_[1], t2_[1]])[..., None]

    out = _tc_final(aggA0, aggA1, aggB0, aggB1, xa, xb, dinv_c, t0, t1c,
                    W1a_, W1b_, b1r, W2, b2r)
    return (out[0].T, out[1].T)


# speedup vs baseline: 39.8579x; 1.4983x over previous
import functools

import jax
import jax.numpy as jnp
from jax import lax
from jax.experimental import pallas as pl
from jax.experimental.pallas import tpu as pltpu
from jax.experimental.pallas import tpu_sc as plsc

N = 100000
E = 3200000
NP = 102400
SLICE = NP // 16
R0 = E // 128
R3 = 25088
RPW = R3 // 16
K = 4
NCH = RPW // K
F32 = jnp.float32

_mesh = plsc.VectorSubcoreMesh(core_axis_name="c", subcore_axis_name="s")
_sc_params = pltpu.CompilerParams(use_tc_tiling_on_sc=False)
_sc_main_params = pltpu.CompilerParams(use_tc_tiling_on_sc=False,
                                       internal_scratch_in_bytes=128 * 1024)


def _sc_deg_body(dst_hbm, zeros1_hbm, ones_hbm, deg_out,
                 deg_sh, dbuf, ones_v, sem):
    cid = lax.axis_index("c")
    sid = lax.axis_index("s")
    nsl = pl.ds(sid * SLICE, SLICE)
    pltpu.sync_copy(zeros1_hbm, deg_sh.at[nsl])
    pltpu.sync_copy(ones_hbm, ones_v)
    plsc.subcore_barrier()

    row0 = sid * RPW

    def step(r, _):
        base = row0 + r * K
        pltpu.sync_copy(dst_hbm.at[cid, pl.ds(base, K)], dbuf)
        for j in range(K):
            pltpu.sync_copy(ones_v, deg_sh.at[dbuf.at[j]], add=True)
        return _

    lax.fori_loop(0, NCH, step, None)
    plsc.subcore_barrier()
    pltpu.sync_copy(deg_sh.at[nsl], deg_out.at[cid, nsl])


def _sc_deg(dsts, zeros1, ones128):
    return pl.kernel(
        _sc_deg_body,
        out_type=jax.ShapeDtypeStruct((2, NP), F32),
        mesh=_mesh,
        compiler_params=_sc_params,
        scratch_types=[
            pltpu.VMEM_SHARED((NP,), F32),
            pltpu.VMEM((K, 128), jnp.int32),
            pltpu.VMEM((128,), F32),
            pltpu.SemaphoreType.DMA,
        ],
    )(dsts, zeros1, ones128)


def _sc_main_body(src_hbm, dst_hbm, ya_hbm, yb_hbm, dinv_hbm, zeros16_hbm,
                  zeros1_hbm, agg_a_out, agg_b_out, t_out,
                  acc_sh, t_sh, dinv_sh, sbuf, dbuf, rbuf, dvbuf, semg, sems):
    cid = lax.axis_index("c")
    sid = lax.axis_index("s")
    nsl = pl.ds(sid * SLICE, SLICE)
    row0 = sid * RPW

    pltpu.sync_copy(dinv_hbm.at[cid, nsl], dinv_sh.at[nsl])
    pltpu.sync_copy(zeros16_hbm, acc_sh.at[nsl])
    pltpu.sync_copy(zeros1_hbm, t_sh.at[nsl])
    plsc.subcore_barrier()

    def step_a(r, _):
        base = row0 + r * K
        pltpu.sync_copy(src_hbm.at[cid, pl.ds(base, K)], sbuf)
        pltpu.sync_copy(dst_hbm.at[cid, pl.ds(base, K)], dbuf)
        gd = [pltpu.async_copy(ya_hbm.at[cid].at[sbuf.at[j]],
                               rbuf.at[pl.ds(j * 128, 128)], semg)
              for j in range(K)]
        gd2 = [pltpu.async_copy(dinv_sh.at[dbuf.at[j]], dvbuf.at[j], sems)
               for j in range(K)]
        for d in gd + gd2:
            d.wait()
        for j in range(K):
            pltpu.sync_copy(rbuf.at[pl.ds(j * 128, 128)],
                            acc_sh.at[dbuf.at[j]], add=True)
            pltpu.sync_copy(dvbuf.at[j], t_sh.at[sbuf.at[j]], add=True)
        return _

    lax.fori_loop(0, NCH, step_a, None)
    plsc.subcore_barrier()
    pltpu.sync_copy(acc_sh.at[nsl], agg_a_out.at[cid, nsl])
    pltpu.sync_copy(t_sh.at[nsl], t_out.at[cid, nsl])
    pltpu.sync_copy(zeros16_hbm, acc_sh.at[nsl])
    plsc.subcore_barrier()

    def step_b(r, _):
        base = row0 + r * K
        pltpu.sync_copy(src_hbm.at[cid, pl.ds(base, K)], sbuf)
        pltpu.sync_copy(dst_hbm.at[cid, pl.ds(base, K)], dbuf)
        gd = [pltpu.async_copy(yb_hbm.at[cid].at[sbuf.at[j]],
                               rbuf.at[pl.ds(j * 128, 128)], semg)
              for j in range(K)]
        for d in gd:
            d.wait()
        for j in range(K):
            pltpu.sync_copy(rbuf.at[pl.ds(j * 128, 128)],
                            acc_sh.at[dbuf.at[j]], add=True)
        return _

    lax.fori_loop(0, NCH, step_b, None)
    plsc.subcore_barrier()
    pltpu.sync_copy(acc_sh.at[nsl], agg_b_out.at[cid, nsl])


def _sc_main(srcs, dsts, ya, yb, dinv, zeros16, zeros1):
    return pl.kernel(
        _sc_main_body,
        out_type=[
            jax.ShapeDtypeStruct((2, NP, 16), F32),
            jax.ShapeDtypeStruct((2, NP, 16), F32),
            jax.ShapeDtypeStruct((2, NP), F32),
        ],
        mesh=_mesh,
        compiler_params=_sc_main_params,
        scratch_types=[
            pltpu.VMEM_SHARED((NP, 16), F32),
            pltpu.VMEM_SHARED((NP,), F32),
            pltpu.VMEM_SHARED((NP,), F32),
            pltpu.VMEM((K, 128), jnp.int32),
            pltpu.VMEM((K, 128), jnp.int32),
            pltpu.VMEM((K * 128, 16), F32),
            pltpu.VMEM((K, 128), F32),
            pltpu.SemaphoreType.DMA,
            pltpu.SemaphoreType.DMA,
        ],
    )(srcs, dsts, ya, yb, dinv, zeros16, zeros1)


def _tc_prep_body(deg_ref, xa_ref, xb_ref, dinv_ref, ya_ref, yb_ref):
    dv = lax.rsqrt(deg_ref[0] + 1.0)
    dinv_ref[0] = dv
    ya_ref[0] = dv * xa_ref[0]
    yb_ref[0] = dv * xb_ref[0]


def _tc_prep(deg, xa, xb):
    R = 1024
    nb = NP // R
    grid = (2, nb)
    col = pl.BlockSpec((1, R, 1), lambda g, i: (g, i, 0))
    mat = pl.BlockSpec((1, R, 16), lambda g, i: (g, i, 0))
    return pl.pallas_call(
        _tc_prep_body,
        grid=grid,
        in_specs=[col, mat, mat],
        out_specs=[col, mat, mat],
        out_shape=[
            jax.ShapeDtypeStruct((2, NP, 1), F32),
            jax.ShapeDtypeStruct((2, NP, 16), F32),
            jax.ShapeDtypeStruct((2, NP, 16), F32),
        ],
    )(deg, xa, xb)


def _tc_final_body(aggA, aggB, xa, xb, dinv, t,
                   W1a, W1b, b1, W2, b2, out_ref, acc, *, nb, rows):
    i = pl.program_id(1)

    @pl.when(i == 0)
    def _():
        acc[...] = jnp.zeros_like(acc)

    dv = dinv[0]
    aa = dv * aggA[0] + (dv * dv) * xa[0]
    ab = dv * aggB[0] + (dv * dv) * xb[0]
    h1 = jnp.maximum(
        jnp.dot(aa, W1a[...], preferred_element_type=F32)
        + jnp.dot(ab, W1b[...], preferred_element_type=F32)
        + b1[...], 0.0)
    rowid = i * rows + lax.broadcasted_iota(jnp.int32, (rows, 1), 0)
    m = jnp.where(rowid < N, 1.0, 0.0)
    c = m * dv * (t[0] + dv)
    acc[...] += jnp.sum(c * h1, axis=0, keepdims=True)

    @pl.when(i == nb - 1)
    def _():
        out_ref[0] = (jnp.dot(acc[...], W2[...], preferred_element_type=F32)
                      * (1.0 / N) + b2[...])


def _tc_final(aggA, aggB, xa, xb, dinv, t, W1a, W1b, b1, W2, b2):
    R = 1024
    nb = NP // R
    grid = (2, nb)
    col = pl.BlockSpec((1, R, 1), lambda g, i: (g, i, 0))
    mat = pl.BlockSpec((1, R, 16), lambda g, i: (g, i, 0))
    full = lambda s: pl.BlockSpec(s, lambda g, i: tuple(0 for _ in s))
    body = functools.partial(_tc_final_body, nb=nb, rows=R)
    return pl.pallas_call(
        body,
        grid=grid,
        in_specs=[mat, mat, mat, mat, col, col,
                  full((16, 64)), full((16, 64)), full((1, 64)),
                  full((64, 32)), full((1, 32))],
        out_specs=pl.BlockSpec((1, 1, 32), lambda g, i: (g, 0, 0)),
        out_shape=jax.ShapeDtypeStruct((2, 1, 32), F32),
        scratch_shapes=[pltpu.VMEM((1, 64), F32)],
    )(aggA, aggB, xa, xb, dinv, t, W1a, W1b, b1, W2, b2)


def _pad_edges(v):
    pad = jnp.full((R3 * 128 - E,), N, dtype=jnp.int32)
    return jnp.concatenate([v, pad]).reshape(R3, 128)


def kernel(features_1, edge_index_1, features_2, edge_index_2, W1, b1, W2, b2):
    f32 = jnp.float32
    xs = [jnp.pad(f, ((0, NP - N), (0, 32 - f.shape[1])))
          for f in (features_1, features_2)]
    xa = jnp.stack([x[:, :16] for x in xs])
    xb = jnp.stack([x[:, 16:] for x in xs])
    srcs = jnp.stack([_pad_edges(edge_index_1[0]), _pad_edges(edge_index_2[0])])
    dsts = jnp.stack([_pad_edges(edge_index_1[1]), _pad_edges(edge_index_2[1])])
    zeros16 = jnp.zeros((SLICE, 16), f32)
    zeros1 = jnp.zeros((SLICE,), f32)
    ones128 = jnp.ones((128,), f32)
    W1p = jnp.pad(W1, ((0, 32 - W1.shape[0]), (0, 0)))
    W1a_, W1b_ = W1p[:16], W1p[16:]
    b1r, b2r = b1.reshape(1, 64), b2.reshape(1, 32)

    deg = _sc_deg(dsts, zeros1, ones128)

    dinv_c, ya, yb = _tc_prep(deg[..., None], xa, xb)
    dinv_flat = dinv_c[..., 0]

    aggA, aggB, t = _sc_main(srcs, dsts, ya, yb, dinv_flat, zeros16, zeros1)

    out = _tc_final(aggA, aggB, xa, xb, dinv_c, t[..., None],
                    W1a_, W1b_, b1r, W2, b2r)
    return (out[0].T, out[1].T)


# speedup vs baseline: 48.2215x; 1.2098x over previous
import functools

import jax
import jax.numpy as jnp
from jax import lax
from jax.experimental import pallas as pl
from jax.experimental.pallas import tpu as pltpu
from jax.experimental.pallas import tpu_sc as plsc

N = 100000
E = 3200000
NP = 102400
SLICE = NP // 16
R0 = E // 128
R3 = 25088
RPW = R3 // 16
K = 4
NCH = RPW // K
F32 = jnp.float32

_mesh = plsc.VectorSubcoreMesh(core_axis_name="c", subcore_axis_name="s")
_sc_params = pltpu.CompilerParams(use_tc_tiling_on_sc=False)
_sc_main_params = pltpu.CompilerParams(use_tc_tiling_on_sc=False,
                                       internal_scratch_in_bytes=128 * 1024)


def _sc_deg_body(dst_hbm, zeros1_hbm, ones_hbm, deg_out,
                 deg_sh, dbuf, ones_v, sem):
    cid = lax.axis_index("c")
    sid = lax.axis_index("s")
    nsl = pl.ds(sid * SLICE, SLICE)
    pltpu.sync_copy(zeros1_hbm, deg_sh.at[nsl])
    pltpu.sync_copy(ones_hbm, ones_v)
    plsc.subcore_barrier()

    row0 = sid * RPW

    def step(r, _):
        base = row0 + r * K
        pltpu.sync_copy(dst_hbm.at[cid, pl.ds(base, K)], dbuf)
        for j in range(K):
            pltpu.sync_copy(ones_v.at[j], deg_sh.at[dbuf.at[j]], add=True)
        return _

    lax.fori_loop(0, NCH, step, None)
    plsc.subcore_barrier()
    pltpu.sync_copy(deg_sh.at[nsl], deg_out.at[cid, nsl])


def _sc_deg(dsts, zeros1, ones128):
    return pl.kernel(
        _sc_deg_body,
        out_type=jax.ShapeDtypeStruct((2, NP), F32),
        mesh=_mesh,
        compiler_params=_sc_params,
        scratch_types=[
            pltpu.VMEM_SHARED((NP,), F32),
            pltpu.VMEM((K, 128), jnp.int32),
            pltpu.VMEM((K, 128), F32),
            pltpu.SemaphoreType.DMA,
        ],
    )(dsts, zeros1, ones128)


def _sc_main_body(src_hbm, dst_hbm, ya_hbm, yb_hbm, dinv_hbm, zeros16_hbm,
                  zeros1_hbm, agg_a_out, agg_b_out, t_out,
                  acc_sh, t_sh, sbuf, dbuf, rbuf, dvbuf,
                  sbuf2, dbuf2, rbuf2, dvbuf2, semg, sems, semg2, sems2):
    cid = lax.axis_index("c")
    sid = lax.axis_index("s")
    nsl = pl.ds(sid * SLICE, SLICE)
    row0 = sid * RPW

    pltpu.sync_copy(zeros16_hbm, acc_sh.at[nsl])
    pltpu.sync_copy(zeros1_hbm, t_sh.at[nsl])
    plsc.subcore_barrier()

    def _load(base, sb, db):
        pltpu.sync_copy(src_hbm.at[cid, pl.ds(base, K)], sb)
        pltpu.sync_copy(dst_hbm.at[cid, pl.ds(base, K)], db)

    def _fire(y_ref, sb, db, rb, dvb, sg, st, with_t):
        gd = [pltpu.async_copy(y_ref.at[cid].at[sb.at[j]],
                               rb.at[pl.ds(j * 128, 128)], sg)
              for j in range(K)]
        if with_t:
            gd += [pltpu.async_copy(dinv_hbm.at[cid].at[db.at[j]],
                                    dvb.at[j], st) for j in range(K)]
        return gd

    def _scatter(sb, db, rb, dvb, with_t):
        for j in range(K):
            pltpu.sync_copy(rb.at[pl.ds(j * 128, 128)],
                            acc_sh.at[db.at[j]], add=True)
            if with_t:
                pltpu.sync_copy(dvb.at[j], t_sh.at[sb.at[j]], add=True)

    def _make_pass(y_ref, with_t):
        def step(rr, _):
            base_a = row0 + (2 * rr) * K
            base_b = base_a + K
            _load(base_a, sbuf, dbuf)
            ga = _fire(y_ref, sbuf, dbuf, rbuf, dvbuf, semg, sems, with_t)
            _load(base_b, sbuf2, dbuf2)
            gb = _fire(y_ref, sbuf2, dbuf2, rbuf2, dvbuf2, semg2, sems2,
                       with_t)
            for d in ga:
                d.wait()
            _scatter(sbuf, dbuf, rbuf, dvbuf, with_t)
            for d in gb:
                d.wait()
            _scatter(sbuf2, dbuf2, rbuf2, dvbuf2, with_t)
            return _
        return step

    lax.fori_loop(0, NCH // 2, _make_pass(ya_hbm, True), None)
    plsc.subcore_barrier()
    pltpu.sync_copy(acc_sh.at[nsl], agg_a_out.at[cid, nsl])
    pltpu.sync_copy(t_sh.at[nsl], t_out.at[cid, nsl])
    pltpu.sync_copy(zeros16_hbm, acc_sh.at[nsl])
    plsc.subcore_barrier()

    lax.fori_loop(0, NCH // 2, _make_pass(yb_hbm, False), None)
    plsc.subcore_barrier()
    pltpu.sync_copy(acc_sh.at[nsl], agg_b_out.at[cid, nsl])


def _sc_main(srcs, dsts, ya, yb, dinv, zeros16, zeros1):
    return pl.kernel(
        _sc_main_body,
        out_type=[
            jax.ShapeDtypeStruct((2, NP, 16), F32),
            jax.ShapeDtypeStruct((2, NP, 16), F32),
            jax.ShapeDtypeStruct((2, NP), F32),
        ],
        mesh=_mesh,
        compiler_params=_sc_main_params,
        scratch_types=[
            pltpu.VMEM_SHARED((NP, 16), F32),
            pltpu.VMEM_SHARED((NP,), F32),
            pltpu.VMEM((K, 128), jnp.int32),
            pltpu.VMEM((K, 128), jnp.int32),
            pltpu.VMEM((K * 128, 16), F32),
            pltpu.VMEM((K, 128), F32),
            pltpu.VMEM((K, 128), jnp.int32),
            pltpu.VMEM((K, 128), jnp.int32),
            pltpu.VMEM((K * 128, 16), F32),
            pltpu.VMEM((K, 128), F32),
            pltpu.SemaphoreType.DMA,
            pltpu.SemaphoreType.DMA,
            pltpu.SemaphoreType.DMA,
            pltpu.SemaphoreType.DMA,
        ],
    )(srcs, dsts, ya, yb, dinv, zeros16, zeros1)


def _tc_prep_body(deg_ref, xa_ref, xb_ref, dinv_ref, ya_ref, yb_ref):
    dv = lax.rsqrt(deg_ref[0] + 1.0)
    dinv_ref[0] = dv
    ya_ref[0] = dv * xa_ref[0]
    yb_ref[0] = dv * xb_ref[0]


def _tc_prep(deg, xa, xb):
    R = 1024
    nb = NP // R
    grid = (2, nb)
    col = pl.BlockSpec((1, R, 1), lambda g, i: (g, i, 0))
    mat = pl.BlockSpec((1, R, 16), lambda g, i: (g, i, 0))
    return pl.pallas_call(
        _tc_prep_body,
        grid=grid,
        in_specs=[col, mat, mat],
        out_specs=[col, mat, mat],
        out_shape=[
            jax.ShapeDtypeStruct((2, NP, 1), F32),
            jax.ShapeDtypeStruct((2, NP, 16), F32),
            jax.ShapeDtypeStruct((2, NP, 16), F32),
        ],
    )(deg, xa, xb)


def _tc_final_body(aggA, aggB, xa, xb, dinv, t,
                   W1a, W1b, b1, W2, b2, out_ref, acc, *, nb, rows):
    i = pl.program_id(1)

    @pl.when(i == 0)
    def _():
        acc[...] = jnp.zeros_like(acc)

    dv = dinv[0]
    aa = dv * aggA[0] + (dv * dv) * xa[0]
    ab = dv * aggB[0] + (dv * dv) * xb[0]
    h1 = jnp.maximum(
        jnp.dot(aa, W1a[...], preferred_element_type=F32)
        + jnp.dot(ab, W1b[...], preferred_element_type=F32)
        + b1[...], 0.0)
    rowid = i * rows + lax.broadcasted_iota(jnp.int32, (rows, 1), 0)
    m = jnp.where(rowid < N, 1.0, 0.0)
    c = m * dv * (t[0] + dv)
    acc[...] += jnp.sum(c * h1, axis=0, keepdims=True)

    @pl.when(i == nb - 1)
    def _():
        out_ref[0] = (jnp.dot(acc[...], W2[...], preferred_element_type=F32)
                      * (1.0 / N) + b2[...])


def _tc_final(aggA, aggB, xa, xb, dinv, t, W1a, W1b, b1, W2, b2):
    R = 1024
    nb = NP // R
    grid = (2, nb)
    col = pl.BlockSpec((1, R, 1), lambda g, i: (g, i, 0))
    mat = pl.BlockSpec((1, R, 16), lambda g, i: (g, i, 0))
    full = lambda s: pl.BlockSpec(s, lambda g, i: tuple(0 for _ in s))
    body = functools.partial(_tc_final_body, nb=nb, rows=R)
    return pl.pallas_call(
        body,
        grid=grid,
        in_specs=[mat, mat, mat, mat, col, col,
                  full((16, 64)), full((16, 64)), full((1, 64)),
                  full((64, 32)), full((1, 32))],
        out_specs=pl.BlockSpec((1, 1, 32), lambda g, i: (g, 0, 0)),
        out_shape=jax.ShapeDtypeStruct((2, 1, 32), F32),
        scratch_shapes=[pltpu.VMEM((1, 64), F32)],
    )(aggA, aggB, xa, xb, dinv, t, W1a, W1b, b1, W2, b2)


def _pad_edges(v):
    pad = jnp.full((R3 * 128 - E,), N, dtype=jnp.int32)
    return jnp.concatenate([v, pad]).reshape(R3, 128)


def kernel(features_1, edge_index_1, features_2, edge_index_2, W1, b1, W2, b2):
    f32 = jnp.float32
    xs = [jnp.pad(f, ((0, NP - N), (0, 32 - f.shape[1])))
          for f in (features_1, features_2)]
    xa = jnp.stack([x[:, :16] for x in xs])
    xb = jnp.stack([x[:, 16:] for x in xs])
    srcs = jnp.stack([_pad_edges(edge_index_1[0]), _pad_edges(edge_index_2[0])])
    dsts = jnp.stack([_pad_edges(edge_index_1[1]), _pad_edges(edge_index_2[1])])
    zeros16 = jnp.zeros((SLICE, 16), f32)
    zeros1 = jnp.zeros((SLICE,), f32)
    ones128 = jnp.ones((K, 128), f32)
    W1p = jnp.pad(W1, ((0, 32 - W1.shape[0]), (0, 0)))
    W1a_, W1b_ = W1p[:16], W1p[16:]
    b1r, b2r = b1.reshape(1, 64), b2.reshape(1, 32)

    deg = _sc_deg(dsts, zeros1, ones128)

    dinv_c, ya, yb = _tc_prep(deg[..., None], xa, xb)
    dinv_flat = dinv_c[..., 0]

    aggA, aggB, t = _sc_main(srcs, dsts, ya, yb, dinv_flat, zeros16, zeros1)

    out = _tc_final(aggA, aggB, xa, xb, dinv_c, t[..., None],
                    W1a_, W1b_, b1r, W2, b2r)
    return (out[0].T, out[1].T)


# speedup vs baseline: 50.5973x; 1.0493x over previous
import functools

import jax
import jax.numpy as jnp
from jax import lax
from jax.experimental import pallas as pl
from jax.experimental.pallas import tpu as pltpu
from jax.experimental.pallas import tpu_sc as plsc

N = 100000
E = 3200000
NP = 102400
SLICE = NP // 16
R3 = 25088
RPW = R3 // 16
K = 4
NCH = RPW // K
CROWS = 128
F32 = jnp.float32

_mesh = plsc.VectorSubcoreMesh(core_axis_name="c", subcore_axis_name="s")
_sc_params = pltpu.CompilerParams(use_tc_tiling_on_sc=False)


def _newton_rsqrt(x):
    i = lax.bitcast_convert_type(x, jnp.int32)
    i = jnp.int32(0x5F3759DF) - lax.shift_right_arithmetic(i, 1)
    y = lax.bitcast_convert_type(i, F32)
    for _ in range(3):
        y = y * (1.5 - 0.5 * x * y * y)
    return y


def _sc_all_body(src_hbm, dst_hbm, xa_hbm, xb_hbm, zeros16_hbm, zeros1_hbm,
                 ones_hbm, agg_a_out, agg_b_out, t_out, dinv_out, ya_out,
                 yb_out, acc_sh, dt_sh, sbuf, dbuf, rbuf, dvbuf,
                 sbuf2, dbuf2, rbuf2, dvbuf2, xbuf, dchunk,
                 semg, sems, semg2, sems2, semi):
    cid = lax.axis_index("c")
    sid = lax.axis_index("s")
    nsl = pl.ds(sid * SLICE, SLICE)
    row0 = sid * RPW

    ones_v = dvbuf
    pltpu.sync_copy(zeros16_hbm, acc_sh.at[nsl])
    pltpu.sync_copy(zeros1_hbm, dt_sh.at[nsl])
    pltpu.sync_copy(ones_hbm, ones_v)
    plsc.subcore_barrier()

    def step_deg(rr, _):
        base_a = row0 + (2 * rr) * K
        la = pltpu.async_copy(dst_hbm.at[cid, pl.ds(base_a, K)], dbuf, semi)
        lb = pltpu.async_copy(dst_hbm.at[cid, pl.ds(base_a + K, K)], dbuf2,
                              semi)
        la.wait()
        for j in range(K):
            pltpu.sync_copy(ones_v.at[j], dt_sh.at[dbuf.at[j]], add=True)
        lb.wait()
        for j in range(K):
            pltpu.sync_copy(ones_v.at[j], dt_sh.at[dbuf2.at[j]], add=True)
        return _

    lax.fori_loop(0, NCH // 2, step_deg, None)
    plsc.subcore_barrier()

    def step_prep(ci, _):
        base = sid * SLICE + ci * CROWS
        csl = pl.ds(base, CROWS)
        pltpu.sync_copy(dt_sh.at[csl], dchunk)
        for i in range(CROWS // 16):
            s16 = pl.ds(i * 16, 16)
            dchunk[s16] = _newton_rsqrt(dchunk[s16] + 1.0)
        pltpu.sync_copy(dchunk, dinv_out.at[cid, csl])

        def scale(x_hbm, y_out):
            pltpu.sync_copy(x_hbm.at[cid, csl], xbuf)

            def rowmul16(i, _):
                dv16 = dchunk[pl.ds(i * 16, 16)]
                for j in range(16):
                    r = i * 16 + j
                    xbuf[r] = dv16[j] * xbuf[r]
                return _

            lax.fori_loop(0, CROWS // 16, rowmul16, None)
            pltpu.sync_copy(xbuf, y_out.at[cid, csl])

        scale(xa_hbm, ya_out)
        scale(xb_hbm, yb_out)
        return _

    lax.fori_loop(0, SLICE // CROWS, step_prep, None)
    pltpu.sync_copy(zeros1_hbm, dt_sh.at[nsl])
    plsc.subcore_barrier()

    def _load(base, sb, db):
        pltpu.sync_copy(src_hbm.at[cid, pl.ds(base, K)], sb)
        pltpu.sync_copy(dst_hbm.at[cid, pl.ds(base, K)], db)

    def _fire(y_ref, sb, db, rb, dvb, sg, st, with_t):
        gd = [pltpu.async_copy(y_ref.at[cid].at[sb.at[j]],
                               rb.at[pl.ds(j * 128, 128)], sg)
              for j in range(K)]
        if with_t:
            gd += [pltpu.async_copy(dinv_out.at[cid].at[db.at[j]],
                                    dvb.at[j], st) for j in range(K)]
        return gd

    def _scatter(sb, db, rb, dvb, with_t):
        for j in range(K):
            pltpu.sync_copy(rb.at[pl.ds(j * 128, 128)],
                            acc_sh.at[db.at[j]], add=True)
            if with_t:
                pltpu.sync_copy(dvb.at[j], dt_sh.at[sb.at[j]], add=True)

    def _make_pass(y_ref, with_t):
        def step(rr, _):
            base_a = row0 + (2 * rr) * K
            base_b = base_a + K
            _load(base_a, sbuf, dbuf)
            ga = _fire(y_ref, sbuf, dbuf, rbuf, dvbuf, semg, sems, with_t)
            _load(base_b, sbuf2, dbuf2)
            gb = _fire(y_ref, sbuf2, dbuf2, rbuf2, dvbuf2, semg2, sems2,
                       with_t)
            for d in ga:
                d.wait()
            _scatter(sbuf, dbuf, rbuf, dvbuf, with_t)
            for d in gb:
                d.wait()
            _scatter(sbuf2, dbuf2, rbuf2, dvbuf2, with_t)
            return _
        return step

    lax.fori_loop(0, NCH // 2, _make_pass(ya_out, True), None)
    plsc.subcore_barrier()
    pltpu.sync_copy(acc_sh.at[nsl], agg_a_out.at[cid, nsl])
    pltpu.sync_copy(dt_sh.at[nsl], t_out.at[cid, nsl])
    pltpu.sync_copy(zeros16_hbm, acc_sh.at[nsl])
    plsc.subcore_barrier()

    lax.fori_loop(0, NCH // 2, _make_pass(yb_out, False), None)
    plsc.subcore_barrier()
    pltpu.sync_copy(acc_sh.at[nsl], agg_b_out.at[cid, nsl])


def _sc_all(srcs, dsts, xa, xb, zeros16, zeros1, ones):
    return pl.kernel(
        _sc_all_body,
        out_type=[
            jax.ShapeDtypeStruct((2, NP, 16), F32),
            jax.ShapeDtypeStruct((2, NP, 16), F32),
            jax.ShapeDtypeStruct((2, NP), F32),
            jax.ShapeDtypeStruct((2, NP), F32),
            jax.ShapeDtypeStruct((2, NP, 16), F32),
            jax.ShapeDtypeStruct((2, NP, 16), F32),
        ],
        mesh=_mesh,
        compiler_params=_sc_params,
        scratch_types=[
            pltpu.VMEM_SHARED((NP, 16), F32),
            pltpu.VMEM_SHARED((NP,), F32),
            pltpu.VMEM((K, 128), jnp.int32),
            pltpu.VMEM((K, 128), jnp.int32),
            pltpu.VMEM((K * 128, 16), F32),
            pltpu.VMEM((K, 128), F32),
            pltpu.VMEM((K, 128), jnp.int32),
            pltpu.VMEM((K, 128), jnp.int32),
            pltpu.VMEM((K * 128, 16), F32),
            pltpu.VMEM((K, 128), F32),
            pltpu.VMEM((CROWS, 16), F32),
            pltpu.VMEM((CROWS,), F32),
            pltpu.SemaphoreType.DMA,
            pltpu.SemaphoreType.DMA,
            pltpu.SemaphoreType.DMA,
            pltpu.SemaphoreType.DMA,
            pltpu.SemaphoreType.DMA,
        ],
    )(srcs, dsts, xa, xb, zeros16, zeros1, ones)


def _tc_final_body(aggA, aggB, xa, xb, dinv, t,
                   W1a, W1b, b1, W2, b2, out_ref, acc, *, nb, rows):
    i = pl.program_id(1)

    @pl.when(i == 0)
    def _():
        acc[...] = jnp.zeros_like(acc)

    dv = dinv[0]
    aa = dv * aggA[0] + (dv * dv) * xa[0]
    ab = dv * aggB[0] + (dv * dv) * xb[0]
    h1 = jnp.maximum(
        jnp.dot(aa, W1a[...], preferred_element_type=F32)
        + jnp.dot(ab, W1b[...], preferred_element_type=F32)
        + b1[...], 0.0)
    rowid = i * rows + lax.broadcasted_iota(jnp.int32, (rows, 1), 0)
    m = jnp.where(rowid < N, 1.0, 0.0)
    c = m * dv * (t[0] + dv)
    acc[...] += jnp.sum(c * h1, axis=0, keepdims=True)

    @pl.when(i == nb - 1)
    def _():
        out_ref[0] = (jnp.dot(acc[...], W2[...], preferred_element_type=F32)
                      * (1.0 / N) + b2[...])


def _tc_final(aggA, aggB, xa, xb, dinv, t, W1a, W1b, b1, W2, b2):
    R = 1024
    nb = NP // R
    grid = (2, nb)
    col = pl.BlockSpec((1, R, 1), lambda g, i: (g, i, 0))
    mat = pl.BlockSpec((1, R, 16), lambda g, i: (g, i, 0))
    full = lambda s: pl.BlockSpec(s, lambda g, i: tuple(0 for _ in s))
    body = functools.partial(_tc_final_body, nb=nb, rows=R)
    return pl.pallas_call(
        body,
        grid=grid,
        in_specs=[mat, mat, mat, mat, col, col,
                  full((16, 64)), full((16, 64)), full((1, 64)),
                  full((64, 32)), full((1, 32))],
        out_specs=pl.BlockSpec((1, 1, 32), lambda g, i: (g, 0, 0)),
        out_shape=jax.ShapeDtypeStruct((2, 1, 32), F32),
        scratch_shapes=[pltpu.VMEM((1, 64), F32)],
    )(aggA, aggB, xa, xb, dinv, t, W1a, W1b, b1, W2, b2)


def _pad_edges(v):
    pad = jnp.full((R3 * 128 - E,), N, dtype=jnp.int32)
    return jnp.concatenate([v, pad]).reshape(R3, 128)


def kernel(features_1, edge_index_1, features_2, edge_index_2, W1, b1, W2, b2):
    f32 = jnp.float32
    xs = [jnp.pad(f, ((0, NP - N), (0, 32 - f.shape[1])))
          for f in (features_1, features_2)]
    xa = jnp.stack([x[:, :16] for x in xs])
    xb = jnp.stack([x[:, 16:] for x in xs])
    srcs = jnp.stack([_pad_edges(edge_index_1[0]), _pad_edges(edge_index_2[0])])
    dsts = jnp.stack([_pad_edges(edge_index_1[1]), _pad_edges(edge_index_2[1])])
    zeros16 = jnp.zeros((SLICE, 16), f32)
    zeros1 = jnp.zeros((SLICE,), f32)
    ones = jnp.ones((K, 128), f32)
    W1p = jnp.pad(W1, ((0, 32 - W1.shape[0]), (0, 0)))
    W1a_, W1b_ = W1p[:16], W1p[16:]
    b1r, b2r = b1.reshape(1, 64), b2.reshape(1, 32)

    aggA, aggB, t, dinv, _ya, _yb = _sc_all(srcs, dsts, xa, xb,
                                            zeros16, zeros1, ones)

    out = _tc_final(aggA, aggB, xa, xb, dinv[..., None], t[..., None],
                    W1a_, W1b_, b1r, W2, b2r)
    return (out[0].T, out[1].T)


# speedup vs baseline: 52.6968x; 1.0415x over previous
import functools

import jax
import jax.numpy as jnp
from jax import lax
from jax.experimental import pallas as pl
from jax.experimental.pallas import tpu as pltpu
from jax.experimental.pallas import tpu_sc as plsc

N = 100000
E = 3200000
NP = 102400
SLICE = NP // 16
R3 = 25088
RPW = R3 // 16
K = 4
NCH = RPW // K
CROWS = 128
F32 = jnp.float32

_mesh = plsc.VectorSubcoreMesh(core_axis_name="c", subcore_axis_name="s")
_sc_params = pltpu.CompilerParams(use_tc_tiling_on_sc=False)


def _newton_rsqrt(x):
    i = lax.bitcast_convert_type(x, jnp.int32)
    i = jnp.int32(0x5F3759DF) - lax.shift_right_arithmetic(i, 1)
    y = lax.bitcast_convert_type(i, F32)
    for _ in range(3):
        y = y * (1.5 - 0.5 * x * y * y)
    return y


def _sc_all_body(src_hbm, dst_hbm, xa_hbm, xb_hbm, zeros1_hbm,
                 ones_hbm, agg_a_out, agg_b_out, t_out, dinv_out, ya_out,
                 yb_out, acc_sh, dt_sh, sbuf, dbuf, rbuf, dvbuf,
                 sbuf2, dbuf2, rbuf2, dvbuf2, xbuf, dchunk,
                 semg, sems, semg2, sems2, semi):
    cid = lax.axis_index("c")
    sid = lax.axis_index("s")
    nsl = pl.ds(sid * SLICE, SLICE)
    row0 = sid * RPW

    ones_v = dvbuf
    pltpu.sync_copy(zeros1_hbm, dt_sh.at[nsl])
    pltpu.sync_copy(ones_hbm, ones_v)
    plsc.subcore_barrier()

    def step_deg(rr, _):
        base_a = row0 + (2 * rr) * K
        la = pltpu.async_copy(dst_hbm.at[cid, pl.ds(base_a, K)], dbuf, semi)
        lb = pltpu.async_copy(dst_hbm.at[cid, pl.ds(base_a + K, K)], dbuf2,
                              semi)
        la.wait()
        for j in range(K):
            pltpu.sync_copy(ones_v.at[j], dt_sh.at[dbuf.at[j]], add=True)
        lb.wait()
        for j in range(K):
            pltpu.sync_copy(ones_v.at[j], dt_sh.at[dbuf2.at[j]], add=True)
        return _

    lax.fori_loop(0, NCH // 2, step_deg, None)
    plsc.subcore_barrier()

    def step_prep(ci, _):
        base = sid * SLICE + ci * CROWS
        csl = pl.ds(base, CROWS)
        pltpu.sync_copy(dt_sh.at[csl], dchunk)
        for i in range(CROWS // 16):
            s16 = pl.ds(i * 16, 16)
            dchunk[s16] = _newton_rsqrt(dchunk[s16] + 1.0)
        pltpu.sync_copy(dchunk, dinv_out.at[cid, csl])

        def scale(x_hbm, y_out):
            pltpu.sync_copy(x_hbm.at[cid, csl], xbuf)

            def rowmul16(i, _):
                dv16 = dchunk[pl.ds(i * 16, 16)]
                for j in range(16):
                    r = i * 16 + j
                    xbuf[r] = dv16[j] * xbuf[r]
                return _

            lax.fori_loop(0, CROWS // 16, rowmul16, None)
            pltpu.sync_copy(xbuf, y_out.at[cid, csl])
            return xbuf

        yab = scale(xa_hbm, ya_out)
        pltpu.sync_copy(yab, acc_sh.at[csl])
        scale(xb_hbm, yb_out)
        return _

    lax.fori_loop(0, SLICE // CROWS, step_prep, None)
    pltpu.sync_copy(zeros1_hbm, dt_sh.at[nsl])
    plsc.subcore_barrier()

    def _load(base, sb, db):
        pltpu.sync_copy(src_hbm.at[cid, pl.ds(base, K)], sb)
        pltpu.sync_copy(dst_hbm.at[cid, pl.ds(base, K)], db)

    def _fire(y_ref, sb, db, rb, dvb, sg, st, with_t):
        gd = [pltpu.async_copy(y_ref.at[cid].at[sb.at[j]],
                               rb.at[pl.ds(j * 128, 128)], sg)
              for j in range(K)]
        if with_t:
            gd += [pltpu.async_copy(dinv_out.at[cid].at[db.at[j]],
                                    dvb.at[j], st) for j in range(K)]
        return gd

    def _scatter(sb, db, rb, dvb, with_t):
        for j in range(K):
            pltpu.sync_copy(rb.at[pl.ds(j * 128, 128)],
                            acc_sh.at[db.at[j]], add=True)
            if with_t:
                pltpu.sync_copy(dvb.at[j], dt_sh.at[sb.at[j]], add=True)

    def _make_pass(y_ref, with_t):
        def step(rr, _):
            base_a = row0 + (2 * rr) * K
            base_b = base_a + K
            _load(base_a, sbuf, dbuf)
            ga = _fire(y_ref, sbuf, dbuf, rbuf, dvbuf, semg, sems, with_t)
            _load(base_b, sbuf2, dbuf2)
            gb = _fire(y_ref, sbuf2, dbuf2, rbuf2, dvbuf2, semg2, sems2,
                       with_t)
            for d in ga:
                d.wait()
            _scatter(sbuf, dbuf, rbuf, dvbuf, with_t)
            for d in gb:
                d.wait()
            _scatter(sbuf2, dbuf2, rbuf2, dvbuf2, with_t)
            return _
        return step

    lax.fori_loop(0, NCH // 2, _make_pass(ya_out, True), None)
    plsc.subcore_barrier()
    pltpu.sync_copy(acc_sh.at[nsl], agg_a_out.at[cid, nsl])
    pltpu.sync_copy(dt_sh.at[nsl], t_out.at[cid, nsl])
    pltpu.sync_copy(yb_out.at[cid, nsl], acc_sh.at[nsl])
    plsc.subcore_barrier()

    lax.fori_loop(0, NCH // 2, _make_pass(yb_out, False), None)
    plsc.subcore_barrier()
    pltpu.sync_copy(acc_sh.at[nsl], agg_b_out.at[cid, nsl])


def _sc_all(srcs, dsts, xa, xb, zeros1, ones):
    return pl.kernel(
        _sc_all_body,
        out_type=[
            jax.ShapeDtypeStruct((2, NP, 16), F32),
            jax.ShapeDtypeStruct((2, NP, 16), F32),
            jax.ShapeDtypeStruct((2, NP), F32),
            jax.ShapeDtypeStruct((2, NP), F32),
            jax.ShapeDtypeStruct((2, NP, 16), F32),
            jax.ShapeDtypeStruct((2, NP, 16), F32),
        ],
        mesh=_mesh,
        compiler_params=_sc_params,
        scratch_types=[
            pltpu.VMEM_SHARED((NP, 16), F32),
            pltpu.VMEM_SHARED((NP,), F32),
            pltpu.VMEM((K, 128), jnp.int32),
            pltpu.VMEM((K, 128), jnp.int32),
            pltpu.VMEM((K * 128, 16), F32),
            pltpu.VMEM((K, 128), F32),
            pltpu.VMEM((K, 128), jnp.int32),
            pltpu.VMEM((K, 128), jnp.int32),
            pltpu.VMEM((K * 128, 16), F32),
            pltpu.VMEM((K, 128), F32),
            pltpu.VMEM((CROWS, 16), F32),
            pltpu.VMEM((CROWS,), F32),
            pltpu.SemaphoreType.DMA,
            pltpu.SemaphoreType.DMA,
            pltpu.SemaphoreType.DMA,
            pltpu.SemaphoreType.DMA,
            pltpu.SemaphoreType.DMA,
        ],
    )(srcs, dsts, xa, xb, zeros1, ones)


def _tc_final_body(aggA, aggB, dinv, t,
                   W1a, W1b, b1, W2, b2, out_ref, acc, *, nb, rows):
    i = pl.program_id(1)

    @pl.when(i == 0)
    def _():
        acc[...] = jnp.zeros_like(acc)

    dv = dinv[0]
    aa = dv * aggA[0]
    ab = dv * aggB[0]
    h1 = jnp.maximum(
        jnp.dot(aa, W1a[...], preferred_element_type=F32)
        + jnp.dot(ab, W1b[...], preferred_element_type=F32)
        + b1[...], 0.0)
    rowid = i * rows + lax.broadcasted_iota(jnp.int32, (rows, 1), 0)
    m = jnp.where(rowid < N, 1.0, 0.0)
    c = m * dv * (t[0] + dv)
    acc[...] += jnp.sum(c * h1, axis=0, keepdims=True)

    @pl.when(i == nb - 1)
    def _():
        out_ref[0] = (jnp.dot(acc[...], W2[...], preferred_element_type=F32)
                      * (1.0 / N) + b2[...])


def _tc_final(aggA, aggB, dinv, t, W1a, W1b, b1, W2, b2):
    R = 2048
    nb = NP // R
    grid = (2, nb)
    col = pl.BlockSpec((1, R, 1), lambda g, i: (g, i, 0))
    mat = pl.BlockSpec((1, R, 16), lambda g, i: (g, i, 0))
    full = lambda s: pl.BlockSpec(s, lambda g, i: tuple(0 for _ in s))
    body = functools.partial(_tc_final_body, nb=nb, rows=R)
    return pl.pallas_call(
        body,
        grid=grid,
        in_specs=[mat, mat, col, col,
                  full((16, 64)), full((16, 64)), full((1, 64)),
                  full((64, 32)), full((1, 32))],
        out_specs=pl.BlockSpec((1, 1, 32), lambda g, i: (g, 0, 0)),
        out_shape=jax.ShapeDtypeStruct((2, 1, 32), F32),
        scratch_shapes=[pltpu.VMEM((1, 64), F32)],
    )(aggA, aggB, dinv, t, W1a, W1b, b1, W2, b2)


def _pad_edges(v):
    pad = jnp.full((R3 * 128 - E,), N, dtype=jnp.int32)
    return jnp.concatenate([v, pad]).reshape(R3, 128)


def kernel(features_1, edge_index_1, features_2, edge_index_2, W1, b1, W2, b2):
    f32 = jnp.float32
    xs = [jnp.pad(f, ((0, NP - N), (0, 32 - f.shape[1])))
          for f in (features_1, features_2)]
    xa = jnp.stack([x[:, :16] for x in xs])
    xb = jnp.stack([x[:, 16:] for x in xs])
    srcs = jnp.stack([_pad_edges(edge_index_1[0]), _pad_edges(edge_index_2[0])])
    dsts = jnp.stack([_pad_edges(edge_index_1[1]), _pad_edges(edge_index_2[1])])
    zeros1 = jnp.zeros((SLICE,), f32)
    ones = jnp.ones((K, 128), f32)
    W1p = jnp.pad(W1, ((0, 32 - W1.shape[0]), (0, 0)))
    W1a_, W1b_ = W1p[:16], W1p[16:]
    b1r, b2r = b1.reshape(1, 64), b2.reshape(1, 32)

    aggA, aggB, t, dinv, _ya, _yb = _sc_all(srcs, dsts, xa, xb, zeros1, ones)

    out = _tc_final(aggA, aggB, dinv[..., None], t[..., None],
                    W1a_, W1b_, b1r, W2, b2r)
    return (out[0].T, out[1].T)
